# baseline (speedup 1.0000x reference)
import functools

import jax
import jax.numpy as jnp
from jax import lax
from jax.experimental import pallas as pl
from jax.experimental.pallas import tpu as pltpu
from jax.experimental.pallas import tpu_sc as plsc

BATCH = 1024
STATE_DIM = 768
LATENT_DIM = 256
NUM_EMB = 1024
HIDDEN = 64

_HI = jax.lax.Precision.HIGHEST

_NC = 2
_NS = 16
_NW = _NC * _NS
_B_PER_W = BATCH // _NW


def _encode_body(x_ref, W1_ref, b1_ref, W2_ref, b2_ref, emb_ref,
                 ze_ref, idx_ref):
    x = x_ref[...]
    h = jnp.maximum(
        jnp.dot(x, W1_ref[...], preferred_element_type=jnp.float32) + b1_ref[...], 0.0)
    z_e = jnp.dot(h, W2_ref[...], preferred_element_type=jnp.float32) + b2_ref[...]
    ze_ref[...] = z_e

    emb = emb_ref[...]
    scores = jax.lax.dot_general(
        z_e, emb, (((1,), (1,)), ((), ())),
        preferred_element_type=jnp.float32, precision=_HI)
    emb_sq = jax.lax.dot_general(
        jnp.ones((1, LATENT_DIM), jnp.float32), emb * emb,
        (((1,), (1,)), ((), ())), preferred_element_type=jnp.float32,
        precision=_HI)
    dist = emb_sq - 2.0 * scores

    iota = jax.lax.broadcasted_iota(jnp.int32, (BATCH, NUM_EMB), 1)
    m1 = jnp.min(dist, axis=1, keepdims=True)
    idx_ref[...] = jnp.min(jnp.where(dist <= m1, iota, NUM_EMB), axis=1,
                           keepdims=True)


def _decode_body(zq_ref, W3_ref, b3_ref, W4_ref, b4_ref, xr_ref):
    h2 = jnp.maximum(
        jnp.dot(zq_ref[...], W3_ref[...], preferred_element_type=jnp.float32)
        + b3_ref[...], 0.0)
    xr_ref[...] = jnp.dot(h2, W4_ref[...], preferred_element_type=jnp.float32) + b4_ref[...]


def _sc_gather_body(emb_hbm, idx_hbm, out_hbm, idx_v, rows_v, sem):
    wid = lax.axis_index("s") * _NC + lax.axis_index("c")
    base = wid * _B_PER_W
    pltpu.sync_copy(idx_hbm.at[pl.ds(base, _B_PER_W)], idx_v)
    pltpu.async_copy(emb_hbm.at[idx_v], rows_v, sem).wait()
    pltpu.sync_copy(rows_v, out_hbm.at[pl.ds(base, _B_PER_W)])


_sc_gather = functools.partial(
    pl.kernel,
    mesh=plsc.VectorSubcoreMesh(core_axis_name="c", subcore_axis_name="s"),
    out_type=jax.ShapeDtypeStruct((BATCH, LATENT_DIM), jnp.float32),
    scratch_types=[
        pltpu.VMEM((_B_PER_W,), jnp.int32),
        pltpu.VMEM((_B_PER_W, LATENT_DIM), jnp.float32),
        pltpu.SemaphoreType.DMA,
    ],
)(_sc_gather_body)


@jax.jit
def kernel(x, W1, b1, W2, b2, emb, W3, b3, W4, b4):
    b1r = b1.reshape(1, HIDDEN)
    b2r = b2.reshape(1, LATENT_DIM)
    b3r = b3.reshape(1, HIDDEN)
    b4r = b4.reshape(1, STATE_DIM)
    full = lambda *_: (0, 0)
    z_e, idx = pl.pallas_call(
        _encode_body,
        grid=(1,),
        in_specs=[
            pl.BlockSpec((BATCH, STATE_DIM), full),
            pl.BlockSpec((STATE_DIM, HIDDEN), full),
            pl.BlockSpec((1, HIDDEN), full),
            pl.BlockSpec((HIDDEN, LATENT_DIM), full),
            pl.BlockSpec((1, LATENT_DIM), full),
            pl.BlockSpec((NUM_EMB, LATENT_DIM), full),
        ],
        out_specs=[
            pl.BlockSpec((BATCH, LATENT_DIM), full),
            pl.BlockSpec((BATCH, 1), full),
        ],
        out_shape=[
            jax.ShapeDtypeStruct((BATCH, LATENT_DIM), jnp.float32),
            jax.ShapeDtypeStruct((BATCH, 1), jnp.int32),
        ],
    )(x, W1, b1r, W2, b2r, emb)

    z_q = _sc_gather(emb, idx.reshape(BATCH))

    x_recon = pl.pallas_call(
        _decode_body,
        grid=(1,),
        in_specs=[
            pl.BlockSpec((BATCH, LATENT_DIM), full),
            pl.BlockSpec((LATENT_DIM, HIDDEN), full),
            pl.BlockSpec((1, HIDDEN), full),
            pl.BlockSpec((HIDDEN, STATE_DIM), full),
            pl.BlockSpec((1, STATE_DIM), full),
        ],
        out_specs=pl.BlockSpec((BATCH, STATE_DIM), full),
        out_shape=jax.ShapeDtypeStruct((BATCH, STATE_DIM), jnp.float32),
    )(z_q, W3, b3r, W4, b4r)
    return (x_recon, z_e, z_q)


# speedup vs baseline: 1.6435x; 1.6435x over previous
import functools

import jax
import jax.numpy as jnp
from jax.experimental import pallas as pl

BATCH = 1024
STATE_DIM = 768
LATENT_DIM = 256
NUM_EMB = 1024
HIDDEN = 64
BLK = 512

_HI = jax.lax.Precision.HIGHEST


def _fused_body(x_ref, W1_ref, b1_ref, W2_ref, b2_ref, emb_ref, ea_ref, eb_ref,
                ec_ref, W3_ref, b3_ref, W4_ref, b4_ref, xr_ref, ze_ref, zq_ref):
    x = x_ref[...]
    h = jnp.maximum(
        jnp.dot(x, W1_ref[...], preferred_element_type=jnp.float32) + b1_ref[...], 0.0)
    z_e = jnp.dot(h, W2_ref[...], preferred_element_type=jnp.float32) + b2_ref[...]
    ze_ref[...] = z_e

    emb = emb_ref[...]
    scores = jax.lax.dot_general(
        z_e, emb, (((1,), (1,)), ((), ())),
        preferred_element_type=jnp.float32, precision=_HI)
    emb_sq = jax.lax.dot_general(
        jnp.ones((1, LATENT_DIM), jnp.float32), emb * emb,
        (((1,), (1,)), ((), ())), preferred_element_type=jnp.float32,
        precision=_HI)
    dist = emb_sq - 2.0 * scores

    iota = jax.lax.broadcasted_iota(jnp.int32, (BLK, NUM_EMB), 1)
    m1 = jnp.min(dist, axis=1, keepdims=True)
    i1 = jnp.min(jnp.where(dist <= m1, iota, NUM_EMB), axis=1, keepdims=True)

    oh = (iota == i1).astype(jnp.float32)
    z_q = (jnp.dot(oh, ea_ref[...], preferred_element_type=jnp.float32)
           + jnp.dot(oh, eb_ref[...], preferred_element_type=jnp.float32)
           + jnp.dot(oh, ec_ref[...], preferred_element_type=jnp.float32))
    zq_ref[...] = z_q

    h2 = jnp.maximum(
        jnp.dot(z_q, W3_ref[...], preferred_element_type=jnp.float32) + b3_ref[...], 0.0)
    xr_ref[...] = jnp.dot(h2, W4_ref[...], preferred_element_type=jnp.float32) + b4_ref[...]


@jax.jit
def kernel(x, W1, b1, W2, b2, emb, W3, b3, W4, b4):
    b1r = b1.reshape(1, HIDDEN)
    b2r = b2.reshape(1, LATENT_DIM)
    b3r = b3.reshape(1, HIDDEN)
    b4r = b4.reshape(1, STATE_DIM)
    ea = jnp.asarray(emb.astype(jnp.bfloat16), jnp.float32)
    r1 = emb - ea
    eb = jnp.asarray(r1.astype(jnp.bfloat16), jnp.float32)
    ec = r1 - eb
    n_blk = BATCH // BLK
    full = lambda *_: (0, 0)
    row = lambda i: (i, 0)
    x_recon, z_e, z_q = pl.pallas_call(
        _fused_body,
        grid=(n_blk,),
        in_specs=[
            pl.BlockSpec((BLK, STATE_DIM), row),
            pl.BlockSpec((STATE_DIM, HIDDEN), full),
            pl.BlockSpec((1, HIDDEN), full),
            pl.BlockSpec((HIDDEN, LATENT_DIM), full),
            pl.BlockSpec((1, LATENT_DIM), full),
            pl.BlockSpec((NUM_EMB, LATENT_DIM), full),
            pl.BlockSpec((NUM_EMB, LATENT_DIM), full),
            pl.BlockSpec((NUM_EMB, LATENT_DIM), full),
            pl.BlockSpec((NUM_EMB, LATENT_DIM), full),
            pl.BlockSpec((LATENT_DIM, HIDDEN), full),
            pl.BlockSpec((1, HIDDEN), full),
            pl.BlockSpec((HIDDEN, STATE_DIM), full),
            pl.BlockSpec((1, STATE_DIM), full),
        ],
        out_specs=[
            pl.BlockSpec((BLK, STATE_DIM), row),
            pl.BlockSpec((BLK, LATENT_DIM), row),
            pl.BlockSpec((BLK, LATENT_DIM), row),
        ],
        out_shape=[
            jax.ShapeDtypeStruct((BATCH, STATE_DIM), jnp.float32),
            jax.ShapeDtypeStruct((BATCH, LATENT_DIM), jnp.float32),
            jax.ShapeDtypeStruct((BATCH, LATENT_DIM), jnp.float32),
        ],
    )(x, W1, b1r, W2, b2r, emb, ea, eb, ec, W3, b3r, W4, b4r)
    return (x_recon, z_e, z_q)


# speedup vs baseline: 1.9470x; 1.1847x over previous
import functools

import jax
import jax.numpy as jnp
from jax.experimental import pallas as pl

BATCH = 1024
STATE_DIM = 768
LATENT_DIM = 256
NUM_EMB = 1024
HIDDEN = 64
BLK = 512

_HI = jax.lax.Precision.HIGHEST


def _fused_body(x_ref, W1_ref, b1_ref, W2_ref, b2_ref, emb_ref, W3_ref, b3_ref,
                W4_ref, b4_ref, xr_ref, ze_ref, zq_ref):
    x = x_ref[...]
    h = jnp.maximum(
        jnp.dot(x, W1_ref[...], preferred_element_type=jnp.float32) + b1_ref[...], 0.0)
    z_e = jnp.dot(h, W2_ref[...], preferred_element_type=jnp.float32) + b2_ref[...]
    ze_ref[...] = z_e

    emb = emb_ref[...]
    scores = jax.lax.dot_general(
        z_e, emb, (((1,), (1,)), ((), ())),
        preferred_element_type=jnp.float32, precision=_HI)
    emb_sq = jax.lax.dot_general(
        jnp.ones((1, LATENT_DIM), jnp.float32), emb * emb,
        (((1,), (1,)), ((), ())), preferred_element_type=jnp.float32,
        precision=_HI)
    dist = emb_sq - 2.0 * scores

    iota = jax.lax.broadcasted_iota(jnp.int32, (BLK, NUM_EMB), 1)
    m1 = jnp.min(dist, axis=1, keepdims=True)
    i1 = jnp.min(jnp.where(dist <= m1, iota, NUM_EMB), axis=1, keepdims=True)

    mask = jnp.uint32(0xFFFF0000)
    ea = jax.lax.bitcast_convert_type(
        jax.lax.bitcast_convert_type(emb, jnp.uint32) & mask, jnp.float32)
    r = emb - ea
    ebc = jax.lax.bitcast_convert_type(
        jax.lax.bitcast_convert_type(r, jnp.uint32) & mask, jnp.float32)
    ec = r - ebc
    oh = (iota == i1).astype(jnp.float32)
    z_q = (jnp.dot(oh, ea, preferred_element_type=jnp.float32)
           + jnp.dot(oh, ebc, preferred_element_type=jnp.float32)
           + jnp.dot(oh, ec, preferred_element_type=jnp.float32))
    zq_ref[...] = z_q

    h2 = jnp.maximum(
        jnp.dot(z_q, W3_ref[...], preferred_element_type=jnp.float32) + b3_ref[...], 0.0)
    xr_ref[...] = jnp.dot(h2, W4_ref[...], preferred_element_type=jnp.float32) + b4_ref[...]


@jax.jit
def kernel(x, W1, b1, W2, b2, emb, W3, b3, W4, b4):
    b1r = b1.reshape(1, HIDDEN)
    b2r = b2.reshape(1, LATENT_DIM)
    b3r = b3.reshape(1, HIDDEN)
    b4r = b4.reshape(1, STATE_DIM)
    n_blk = BATCH // BLK
    full = lambda *_: (0, 0)
    row = lambda i: (i, 0)
    x_recon, z_e, z_q = pl.pallas_call(
        _fused_body,
        grid=(n_blk,),
        in_specs=[
            pl.BlockSpec((BLK, STATE_DIM), row),
            pl.BlockSpec((STATE_DIM, HIDDEN), full),
            pl.BlockSpec((1, HIDDEN), full),
            pl.BlockSpec((HIDDEN, LATENT_DIM), full),
            pl.BlockSpec((1, LATENT_DIM), full),
            pl.BlockSpec((NUM_EMB, LATENT_DIM), full),
            pl.BlockSpec((LATENT_DIM, HIDDEN), full),
            pl.BlockSpec((1, HIDDEN), full),
            pl.BlockSpec((HIDDEN, STATE_DIM), full),
            pl.BlockSpec((1, STATE_DIM), full),
        ],
        out_specs=[
            pl.BlockSpec((BLK, STATE_DIM), row),
            pl.BlockSpec((BLK, LATENT_DIM), row),
            pl.BlockSpec((BLK, LATENT_DIM), row),
        ],
        out_shape=[
            jax.ShapeDtypeStruct((BATCH, STATE_DIM), jnp.float32),
            jax.ShapeDtypeStruct((BATCH, LATENT_DIM), jnp.float32),
            jax.ShapeDtypeStruct((BATCH, LATENT_DIM), jnp.float32),
        ],
    )(x, W1, b1r, W2, b2r, emb, W3, b3r, W4, b4r)
    return (x_recon, z_e, z_q)
